# Initial kernel scaffold; baseline (speedup 1.0000x reference)
import jax
import jax.numpy as jnp
from jax.experimental import pallas as pl


def kernel(x, adj, W1, b1, W2, b2):
    raise NotImplementedError("write your pallas kernel here")


# speedup vs baseline: 10.6285x; 10.6285x over previous
import functools

import jax
import jax.numpy as jnp
from jax import lax
from jax.experimental import pallas as pl
from jax.experimental.pallas import tpu as pltpu
from jax.experimental.pallas import tpu_sc as plsc

N = 10000
E = 320000
F1 = 128
F2 = 64
NC = 2
NS = 16
NW = NC * NS
B = 128
CH = 80
EPT = B * CH
E_PAD = EPT * NW
NP = 10240
RPT = NP // NS
DEGW = 16
BR = 1024
GR = NP // BR


def _sc_mesh():
    return plsc.VectorSubcoreMesh(
        core_axis_name="c", subcore_axis_name="s", num_cores=NC, num_subcores=NS
    )


@functools.partial(
    pl.kernel,
    out_type=jax.ShapeDtypeStruct((NC, NP, DEGW), jnp.float32),
    mesh=_sc_mesh(),
    compiler_params=pltpu.CompilerParams(use_tc_tiling_on_sc=False),
    scratch_types=[
        pltpu.VMEM((CH, B), jnp.int32),
        pltpu.VMEM((B, DEGW), jnp.float32),
        pltpu.VMEM_SHARED((NP, DEGW), jnp.float32),
    ],
)
def _deg_kernel(dst_hbm, out_hbm, dst_v, ones_v, acc):
    c = lax.axis_index("c")
    s = lax.axis_index("s")
    wid = s * NC + c
    base = s * RPT

    def fill(val):
        def body(i, carry):
            ones_v[i] = jnp.full((DEGW,), val, jnp.float32)
            return carry
        lax.fori_loop(0, B, body, 0)

    fill(0.0)
    for t in range(RPT // B):
        pltpu.sync_copy(ones_v, acc.at[pl.ds(base + t * B, B)])
    fill(1.0)
    plsc.subcore_barrier()

    pltpu.sync_copy(dst_hbm.at[pl.ds(wid * CH, CH)], dst_v)

    def chunk(j, carry):
        pltpu.sync_copy(ones_v, acc.at[dst_v.at[j]], add=True)
        return carry

    lax.fori_loop(0, CH, chunk, 0)
    plsc.subcore_barrier()
    for t in range(RPT // B):
        pltpu.sync_copy(
            acc.at[pl.ds(base + t * B, B)], out_hbm.at[c, pl.ds(base + t * B, B)]
        )


def _make_agg(F):
    @functools.partial(
        pl.kernel,
        out_type=jax.ShapeDtypeStruct((NC, NP, F), jnp.float32),
        mesh=_sc_mesh(),
        compiler_params=pltpu.CompilerParams(use_tc_tiling_on_sc=False),
        scratch_types=[
            pltpu.VMEM((CH, B), jnp.int32),
            pltpu.VMEM((CH, B), jnp.int32),
            pltpu.VMEM((B, F), jnp.float32),
            pltpu.VMEM_SHARED((NP, F), jnp.float32),
            pltpu.SemaphoreType.DMA,
        ],
    )
    def agg(h_hbm, src_hbm, dst_hbm, out_hbm, src_v, dst_v, rows_v, acc, sem):
        c = lax.axis_index("c")
        s = lax.axis_index("s")
        wid = s * NC + c
        base = s * RPT

        def zero_rows(i, carry):
            for k in range(F // 16):
                rows_v[i, pl.ds(k * 16, 16)] = jnp.zeros((16,), jnp.float32)
            return carry

        lax.fori_loop(0, B, zero_rows, 0)
        for t in range(RPT // B):
            pltpu.sync_copy(rows_v, acc.at[pl.ds(base + t * B, B)])
        plsc.subcore_barrier()

        pltpu.sync_copy(src_hbm.at[pl.ds(wid * CH, CH)], src_v)
        pltpu.sync_copy(dst_hbm.at[pl.ds(wid * CH, CH)], dst_v)

        def chunk(j, carry):
            pltpu.async_copy(h_hbm.at[src_v.at[j]], rows_v, sem).wait()
            pltpu.sync_copy(rows_v, acc.at[dst_v.at[j]], add=True)
            return carry

        lax.fori_loop(0, CH, chunk, 0)
        plsc.subcore_barrier()
        for t in range(RPT // B):
            pltpu.sync_copy(
                acc.at[pl.ds(base + t * B, B)], out_hbm.at[c, pl.ds(base + t * B, B)]
            )

    return agg


_agg_l1 = _make_agg(F1)
_agg_l2 = _make_agg(F2)


def _dinv(deg_ref):
    d = deg_ref[0, :, :1] + deg_ref[1, :, :1] + 1.0
    return lax.rsqrt(d)


def _tc_pre1(degp, x_pad, W1):
    def body(deg_ref, x_ref, w_ref, o_ref):
        h = lax.dot_general(
            x_ref[...], w_ref[...], (((1,), (1,)), ((), ())),
            preferred_element_type=jnp.float32,
        )
        o_ref[...] = h * _dinv(deg_ref)

    return pl.pallas_call(
        body,
        grid=(GR,),
        in_specs=[
            pl.BlockSpec((NC, BR, DEGW), lambda i: (0, i, 0)),
            pl.BlockSpec((BR, F1), lambda i: (i, 0)),
            pl.BlockSpec((F1, F1), lambda i: (0, 0)),
        ],
        out_specs=pl.BlockSpec((BR, F1), lambda i: (i, 0)),
        out_shape=jax.ShapeDtypeStruct((NP, F1), jnp.float32),
    )(degp, x_pad, W1)


def _tc_mid(degp, p, h1, b1, W2):
    def body(deg_ref, p_ref, h1_ref, b1_ref, w_ref, o_ref):
        dinv = _dinv(deg_ref)
        g = (p_ref[0] + p_ref[1] + h1_ref[...]) * dinv + b1_ref[...]
        g = jnp.maximum(g, 0.0)
        h = lax.dot_general(
            g, w_ref[...], (((1,), (1,)), ((), ())),
            preferred_element_type=jnp.float32,
        )
        o_ref[...] = h * dinv

    return pl.pallas_call(
        body,
        grid=(GR,),
        in_specs=[
            pl.BlockSpec((NC, BR, DEGW), lambda i: (0, i, 0)),
            pl.BlockSpec((NC, BR, F1), lambda i: (0, i, 0)),
            pl.BlockSpec((BR, F1), lambda i: (i, 0)),
            pl.BlockSpec((1, F1), lambda i: (0, 0)),
            pl.BlockSpec((F2, F1), lambda i: (0, 0)),
        ],
        out_specs=pl.BlockSpec((BR, F2), lambda i: (i, 0)),
        out_shape=jax.ShapeDtypeStruct((NP, F2), jnp.float32),
    )(degp, p, h1, b1, W2)


def _tc_post(degp, q, h2, b2):
    def body(deg_ref, q_ref, h2_ref, b2_ref, o_ref):
        dinv = _dinv(deg_ref)
        g = (q_ref[0] + q_ref[1] + h2_ref[...]) * dinv + b2_ref[...]
        o_ref[...] = jnp.maximum(g, 0.0)

    return pl.pallas_call(
        body,
        grid=(GR,),
        in_specs=[
            pl.BlockSpec((NC, BR, DEGW), lambda i: (0, i, 0)),
            pl.BlockSpec((NC, BR, F2), lambda i: (0, i, 0)),
            pl.BlockSpec((BR, F2), lambda i: (i, 0)),
            pl.BlockSpec((1, F2), lambda i: (0, 0)),
        ],
        out_specs=pl.BlockSpec((BR, F2), lambda i: (i, 0)),
        out_shape=jax.ShapeDtypeStruct((NP, F2), jnp.float32),
    )(degp, q, h2, b2)


def kernel(x, adj, W1, b1, W2, b2):
    src = adj[0].astype(jnp.int32)
    dst = adj[1].astype(jnp.int32)
    pad = jnp.full((E_PAD - E,), N, jnp.int32)
    srcp = jnp.concatenate([src, pad]).reshape(NW * CH, B)
    dstp = jnp.concatenate([dst, pad]).reshape(NW * CH, B)
    x_pad = jnp.pad(x, ((0, NP - N), (0, 0)))

    degp = _deg_kernel(dstp)
    h1 = _tc_pre1(degp, x_pad, W1)
    p = _agg_l1(h1, srcp, dstp)
    h2 = _tc_mid(degp, p, h1, b1.reshape(1, F1), W2)
    q = _agg_l2(h2, srcp, dstp)
    out = _tc_post(degp, q, h2, b2.reshape(1, F2))
    return out[:N]


# speedup vs baseline: 11.4567x; 1.0779x over previous
import functools

import jax
import jax.numpy as jnp
from jax import lax
from jax.experimental import pallas as pl
from jax.experimental.pallas import tpu as pltpu
from jax.experimental.pallas import tpu_sc as plsc

N = 10000
E = 320000
F1 = 128
F2 = 64
NC = 2
NS = 16
NW = NC * NS
B = 128
CH = 80
EPT = B * CH
E_PAD = EPT * NW
NP = 10240
RPT = NP // NS
DEGW = 16
NBUF = 2
AHEAD = 1
HC = CH // 2
BR = 1024
GR = NP // BR


def _sc_mesh():
    return plsc.VectorSubcoreMesh(
        core_axis_name="c", subcore_axis_name="s", num_cores=NC, num_subcores=NS
    )


@functools.partial(
    pl.kernel,
    out_type=jax.ShapeDtypeStruct((NC, NP, DEGW), jnp.float32),
    mesh=_sc_mesh(),
    compiler_params=pltpu.CompilerParams(use_tc_tiling_on_sc=False),
    scratch_types=[
        pltpu.VMEM((CH, B), jnp.int32),
        pltpu.VMEM((B, DEGW), jnp.float32),
        pltpu.VMEM_SHARED((NP, DEGW), jnp.float32),
    ],
)
def _deg_kernel(dst_hbm, out_hbm, dst_v, ones_v, acc):
    c = lax.axis_index("c")
    s = lax.axis_index("s")
    wid = s * NC + c
    base = s * RPT

    def fill(val):
        def body(i, carry):
            ones_v[i] = jnp.full((DEGW,), val, jnp.float32)
            return carry
        lax.fori_loop(0, B, body, 0)

    fill(0.0)
    for t in range(RPT // B):
        pltpu.sync_copy(ones_v, acc.at[pl.ds(base + t * B, B)])
    fill(1.0)
    plsc.subcore_barrier()

    pltpu.sync_copy(dst_hbm.at[pl.ds(wid * CH, CH)], dst_v)

    def chunk(j, carry):
        pltpu.sync_copy(ones_v, acc.at[dst_v.at[j]], add=True)
        return carry

    lax.fori_loop(0, CH, chunk, 0)
    plsc.subcore_barrier()
    for t in range(RPT // B):
        pltpu.sync_copy(
            acc.at[pl.ds(base + t * B, B)], out_hbm.at[c, pl.ds(base + t * B, B)]
        )


def _make_agg(F):
    @functools.partial(
        pl.kernel,
        out_type=jax.ShapeDtypeStruct((NC, NP, F), jnp.float32),
        mesh=_sc_mesh(),
        compiler_params=pltpu.CompilerParams(use_tc_tiling_on_sc=False),
        scratch_types=[
            pltpu.VMEM((HC, B), jnp.int32),
            pltpu.VMEM((HC, B), jnp.int32),
            pltpu.VMEM((NBUF, B, F), jnp.float32),
            pltpu.VMEM_SHARED((NP, F), jnp.float32),
            pltpu.SemaphoreType.DMA,
        ],
    )
    def agg(h_hbm, src_hbm, dst_hbm, out_hbm, src_v, dst_v, rows_v, acc, sem):
        c = lax.axis_index("c")
        s = lax.axis_index("s")
        wid = s * NC + c
        base = s * RPT

        def zero_rows(i, carry):
            for k in range(F // 16):
                rows_v[0, i, pl.ds(k * 16, 16)] = jnp.zeros((16,), jnp.float32)
            return carry

        lax.fori_loop(0, B, zero_rows, 0)
        for t in range(RPT // B):
            pltpu.sync_copy(rows_v.at[0], acc.at[pl.ds(base + t * B, B)])
        plsc.subcore_barrier()

        pltpu.sync_copy(src_hbm.at[pl.ds(wid * CH, HC)], src_v)
        pltpu.sync_copy(dst_hbm.at[pl.ds(wid * CH, HC)], dst_v)

        def start_gather(j):
            pltpu.async_copy(h_hbm.at[src_v.at[j % HC]], rows_v.at[j % NBUF], sem)

        for j in range(AHEAD):
            start_gather(j)

        def chunk(j, carry):
            slot = j % NBUF
            pltpu.make_async_copy(
                h_hbm.at[src_v.at[j % HC]], rows_v.at[slot], sem
            ).wait()

            nxt = j + AHEAD

            @pl.when((nxt < CH) & (nxt % HC == 0))
            def _():
                pltpu.sync_copy(
                    src_hbm.at[pl.ds(wid * CH + (nxt // HC) * HC, HC)], src_v
                )

            @pl.when(nxt < CH)
            def _():
                start_gather(nxt)

            @pl.when((j % HC == 0) & (j > 0))
            def _():
                pltpu.sync_copy(
                    dst_hbm.at[pl.ds(wid * CH + (j // HC) * HC, HC)], dst_v
                )

            pltpu.sync_copy(rows_v.at[slot], acc.at[dst_v.at[j % HC]], add=True)
            return carry

        lax.fori_loop(0, CH, chunk, 0)
        plsc.subcore_barrier()
        for t in range(RPT // B):
            pltpu.sync_copy(
                acc.at[pl.ds(base + t * B, B)], out_hbm.at[c, pl.ds(base + t * B, B)]
            )

    return agg


_agg_l1 = _make_agg(F1)
_agg_l2 = _make_agg(F2)


def _dinv(deg_ref):
    d = deg_ref[0, :, :1] + deg_ref[1, :, :1] + 1.0
    return lax.rsqrt(d)


def _tc_pre1(degp, x_pad, W1):
    def body(deg_ref, x_ref, w_ref, o_ref):
        h = lax.dot_general(
            x_ref[...], w_ref[...], (((1,), (1,)), ((), ())),
            preferred_element_type=jnp.float32,
        )
        o_ref[...] = h * _dinv(deg_ref)

    return pl.pallas_call(
        body,
        grid=(GR,),
        in_specs=[
            pl.BlockSpec((NC, BR, DEGW), lambda i: (0, i, 0)),
            pl.BlockSpec((BR, F1), lambda i: (i, 0)),
            pl.BlockSpec((F1, F1), lambda i: (0, 0)),
        ],
        out_specs=pl.BlockSpec((BR, F1), lambda i: (i, 0)),
        out_shape=jax.ShapeDtypeStruct((NP, F1), jnp.float32),
    )(degp, x_pad, W1)


def _tc_mid(degp, p, h1, b1, W2):
    def body(deg_ref, p_ref, h1_ref, b1_ref, w_ref, o_ref):
        dinv = _dinv(deg_ref)
        g = (p_ref[0] + p_ref[1] + h1_ref[...]) * dinv + b1_ref[...]
        g = jnp.maximum(g, 0.0)
        h = lax.dot_general(
            g, w_ref[...], (((1,), (1,)), ((), ())),
            preferred_element_type=jnp.float32,
        )
        o_ref[...] = h * dinv

    return pl.pallas_call(
        body,
        grid=(GR,),
        in_specs=[
            pl.BlockSpec((NC, BR, DEGW), lambda i: (0, i, 0)),
            pl.BlockSpec((NC, BR, F1), lambda i: (0, i, 0)),
            pl.BlockSpec((BR, F1), lambda i: (i, 0)),
            pl.BlockSpec((1, F1), lambda i: (0, 0)),
            pl.BlockSpec((F2, F1), lambda i: (0, 0)),
        ],
        out_specs=pl.BlockSpec((BR, F2), lambda i: (i, 0)),
        out_shape=jax.ShapeDtypeStruct((NP, F2), jnp.float32),
    )(degp, p, h1, b1, W2)


def _tc_post(degp, q, h2, b2):
    def body(deg_ref, q_ref, h2_ref, b2_ref, o_ref):
        dinv = _dinv(deg_ref)
        g = (q_ref[0] + q_ref[1] + h2_ref[...]) * dinv + b2_ref[...]
        o_ref[...] = jnp.maximum(g, 0.0)

    return pl.pallas_call(
        body,
        grid=(GR,),
        in_specs=[
            pl.BlockSpec((NC, BR, DEGW), lambda i: (0, i, 0)),
            pl.BlockSpec((NC, BR, F2), lambda i: (0, i, 0)),
            pl.BlockSpec((BR, F2), lambda i: (i, 0)),
            pl.BlockSpec((1, F2), lambda i: (0, 0)),
        ],
        out_specs=pl.BlockSpec((BR, F2), lambda i: (i, 0)),
        out_shape=jax.ShapeDtypeStruct((NP, F2), jnp.float32),
    )(degp, q, h2, b2)


def kernel(x, adj, W1, b1, W2, b2):
    src = adj[0].astype(jnp.int32)
    dst = adj[1].astype(jnp.int32)
    pad = jnp.full((E_PAD - E,), N, jnp.int32)
    srcp = jnp.concatenate([src, pad]).reshape(NW * CH, B)
    dstp = jnp.concatenate([dst, pad]).reshape(NW * CH, B)
    x_pad = jnp.pad(x, ((0, NP - N), (0, 0)))

    degp = _deg_kernel(dstp)
    h1 = _tc_pre1(degp, x_pad, W1)
    p = _agg_l1(h1, srcp, dstp)
    h2 = _tc_mid(degp, p, h1, b1.reshape(1, F1), W2)
    q = _agg_l2(h2, srcp, dstp)
    out = _tc_post(degp, q, h2, b2.reshape(1, F2))
    return out[:N]


# speedup vs baseline: 15.4374x; 1.3475x over previous
import functools

import jax
import jax.numpy as jnp
from jax import lax
from jax.experimental import pallas as pl
from jax.experimental.pallas import tpu as pltpu
from jax.experimental.pallas import tpu_sc as plsc

N = 10000
E = 320000
F1 = 128
F2 = 64
NC = 2
NS = 16
NW = NC * NS
NP = 10240
RPT = NP // NS
DEGW = 16
B = 96
CH = 106
HC = CH // 2
E_PAD = B * CH * NW
NBUF = 3
AHEAD = 2
DB = 128
DCH = 80
DE_PAD = DB * DCH * NW
BR = 1024
GR = NP // BR


def _sc_mesh():
    return plsc.VectorSubcoreMesh(
        core_axis_name="c", subcore_axis_name="s", num_cores=NC, num_subcores=NS
    )


@functools.partial(
    pl.kernel,
    out_type=jax.ShapeDtypeStruct((NC, NP, DEGW), jnp.float32),
    mesh=_sc_mesh(),
    compiler_params=pltpu.CompilerParams(use_tc_tiling_on_sc=False),
    scratch_types=[
        pltpu.VMEM((DCH, DB), jnp.int32),
        pltpu.VMEM((DB, DEGW), jnp.float32),
        pltpu.VMEM_SHARED((NP, DEGW), jnp.float32),
    ],
)
def _deg_kernel(dst_hbm, out_hbm, dst_v, ones_v, acc):
    c = lax.axis_index("c")
    s = lax.axis_index("s")
    wid = s * NC + c
    base = s * RPT

    def fill(val):
        def body(i, carry):
            ones_v[i] = jnp.full((DEGW,), val, jnp.float32)
            return carry
        lax.fori_loop(0, DB, body, 0)

    fill(0.0)
    for t in range(RPT // DB):
        pltpu.sync_copy(ones_v, acc.at[pl.ds(base + t * DB, DB)])
    fill(1.0)
    plsc.subcore_barrier()

    pltpu.sync_copy(dst_hbm.at[pl.ds(wid * DCH, DCH)], dst_v)

    def chunk(j, carry):
        pltpu.sync_copy(ones_v, acc.at[dst_v.at[j]], add=True)
        return carry

    lax.fori_loop(0, DCH, chunk, 0)
    plsc.subcore_barrier()
    for t in range(RPT // DB):
        pltpu.sync_copy(
            acc.at[pl.ds(base + t * DB, DB)], out_hbm.at[c, pl.ds(base + t * DB, DB)]
        )


def _make_agg(F):
    @functools.partial(
        pl.kernel,
        out_type=jax.ShapeDtypeStruct((NC, NP, F), jnp.float32),
        mesh=_sc_mesh(),
        compiler_params=pltpu.CompilerParams(use_tc_tiling_on_sc=False),
        scratch_types=[
            pltpu.VMEM((HC, B), jnp.int32),
            pltpu.VMEM((HC, B), jnp.int32),
            pltpu.VMEM((NBUF, B, F), jnp.float32),
            pltpu.VMEM_SHARED((NP, F), jnp.float32),
            pltpu.SemaphoreType.DMA,
        ],
    )
    def agg(h_hbm, src_hbm, dst_hbm, out_hbm, src_v, dst_v, rows_v, acc, sem):
        c = lax.axis_index("c")
        s = lax.axis_index("s")
        wid = s * NC + c
        base = s * RPT

        def zero_rows(i, carry):
            for k in range(F // 16):
                rows_v[0, i, pl.ds(k * 16, 16)] = jnp.zeros((16,), jnp.float32)
            return carry

        lax.fori_loop(0, B, zero_rows, 0)
        nfull, tail = RPT // B, RPT % B
        for t in range(nfull):
            pltpu.sync_copy(rows_v.at[0], acc.at[pl.ds(base + t * B, B)])
        if tail:
            pltpu.sync_copy(
                rows_v.at[0].at[pl.ds(0, tail)], acc.at[pl.ds(base + nfull * B, tail)]
            )
        plsc.subcore_barrier()

        pltpu.sync_copy(src_hbm.at[pl.ds(wid * CH, HC)], src_v)
        pltpu.sync_copy(dst_hbm.at[pl.ds(wid * CH, HC)], dst_v)

        def start_gather(j):
            pltpu.async_copy(h_hbm.at[src_v.at[j % HC]], rows_v.at[j % NBUF], sem)

        for j in range(AHEAD):
            start_gather(j)

        def chunk(j, carry):
            slot = j % NBUF
            pltpu.make_async_copy(
                h_hbm.at[src_v.at[j % HC]], rows_v.at[slot], sem
            ).wait()

            nxt = j + AHEAD

            @pl.when((nxt < CH) & (nxt % HC == 0))
            def _():
                pltpu.sync_copy(
                    src_hbm.at[pl.ds(wid * CH + (nxt // HC) * HC, HC)], src_v
                )

            @pl.when(nxt < CH)
            def _():
                start_gather(nxt)

            @pl.when((j % HC == 0) & (j > 0))
            def _():
                pltpu.sync_copy(
                    dst_hbm.at[pl.ds(wid * CH + (j // HC) * HC, HC)], dst_v
                )

            pltpu.sync_copy(rows_v.at[slot], acc.at[dst_v.at[j % HC]], add=True)
            return carry

        lax.fori_loop(0, CH, chunk, 0)
        plsc.subcore_barrier()
        for t in range(nfull):
            pltpu.sync_copy(
                acc.at[pl.ds(base + t * B, B)], out_hbm.at[c, pl.ds(base + t * B, B)]
            )
        if tail:
            pltpu.sync_copy(
                acc.at[pl.ds(base + nfull * B, tail)],
                out_hbm.at[c, pl.ds(base + nfull * B, tail)],
            )

    return agg


_agg_l1 = _make_agg(F1)
_agg_l2 = _make_agg(F2)


def _dinv(deg_ref):
    d = deg_ref[0, :, :1] + deg_ref[1, :, :1] + 1.0
    return lax.rsqrt(d)


def _tc_pre1(degp, x_pad, W1):
    def body(deg_ref, x_ref, w_ref, o_ref):
        h = lax.dot_general(
            x_ref[...], w_ref[...], (((1,), (1,)), ((), ())),
            preferred_element_type=jnp.float32,
        )
        o_ref[...] = h * _dinv(deg_ref)

    return pl.pallas_call(
        body,
        grid=(GR,),
        in_specs=[
            pl.BlockSpec((NC, BR, DEGW), lambda i: (0, i, 0)),
            pl.BlockSpec((BR, F1), lambda i: (i, 0)),
            pl.BlockSpec((F1, F1), lambda i: (0, 0)),
        ],
        out_specs=pl.BlockSpec((BR, F1), lambda i: (i, 0)),
        out_shape=jax.ShapeDtypeStruct((NP, F1), jnp.float32),
    )(degp, x_pad, W1)


def _tc_mid(degp, p, h1, b1, W2):
    def body(deg_ref, p_ref, h1_ref, b1_ref, w_ref, o_ref):
        dinv = _dinv(deg_ref)
        g = (p_ref[0] + p_ref[1] + h1_ref[...]) * dinv + b1_ref[...]
        g = jnp.maximum(g, 0.0)
        h = lax.dot_general(
            g, w_ref[...], (((1,), (1,)), ((), ())),
            preferred_element_type=jnp.float32,
        )
        o_ref[...] = h * dinv

    return pl.pallas_call(
        body,
        grid=(GR,),
        in_specs=[
            pl.BlockSpec((NC, BR, DEGW), lambda i: (0, i, 0)),
            pl.BlockSpec((NC, BR, F1), lambda i: (0, i, 0)),
            pl.BlockSpec((BR, F1), lambda i: (i, 0)),
            pl.BlockSpec((1, F1), lambda i: (0, 0)),
            pl.BlockSpec((F2, F1), lambda i: (0, 0)),
        ],
        out_specs=pl.BlockSpec((BR, F2), lambda i: (i, 0)),
        out_shape=jax.ShapeDtypeStruct((NP, F2), jnp.float32),
    )(degp, p, h1, b1, W2)


def _tc_post(degp, q, h2, b2):
    def body(deg_ref, q_ref, h2_ref, b2_ref, o_ref):
        dinv = _dinv(deg_ref)
        g = (q_ref[0] + q_ref[1] + h2_ref[...]) * dinv + b2_ref[...]
        o_ref[...] = jnp.maximum(g, 0.0)

    return pl.pallas_call(
        body,
        grid=(GR,),
        in_specs=[
            pl.BlockSpec((NC, BR, DEGW), lambda i: (0, i, 0)),
            pl.BlockSpec((NC, BR, F2), lambda i: (0, i, 0)),
            pl.BlockSpec((BR, F2), lambda i: (i, 0)),
            pl.BlockSpec((1, F2), lambda i: (0, 0)),
        ],
        out_specs=pl.BlockSpec((BR, F2), lambda i: (i, 0)),
        out_shape=jax.ShapeDtypeStruct((NP, F2), jnp.float32),
    )(degp, q, h2, b2)


def kernel(x, adj, W1, b1, W2, b2):
    src = adj[0].astype(jnp.int32)
    dst = adj[1].astype(jnp.int32)
    pad = jnp.full((E_PAD - E,), N, jnp.int32)
    srcp = jnp.concatenate([src, pad]).reshape(NW * CH, B)
    dstp = jnp.concatenate([dst, pad]).reshape(NW * CH, B)
    dpad = jnp.full((DE_PAD - E,), N, jnp.int32)
    dstp_deg = jnp.concatenate([dst, dpad]).reshape(NW * DCH, DB)
    x_pad = jnp.pad(x, ((0, NP - N), (0, 0)))

    degp = _deg_kernel(dstp_deg)
    h1 = _tc_pre1(degp, x_pad, W1)
    p = _agg_l1(h1, srcp, dstp)
    h2 = _tc_mid(degp, p, h1, b1.reshape(1, F1), W2)
    q = _agg_l2(h2, srcp, dstp)
    out = _tc_post(degp, q, h2, b2.reshape(1, F2))
    return out[:N]


# speedup vs baseline: 16.2647x; 1.0536x over previous
import functools

import jax
import jax.numpy as jnp
from jax import lax
from jax.experimental import pallas as pl
from jax.experimental.pallas import tpu as pltpu
from jax.experimental.pallas import tpu_sc as plsc

N = 10000
E = 320000
F1 = 128
F2 = 64
NC = 2
NS = 16
NW = NC * NS
NP = 10240
RPT = NP // NS
DEGW = 16
B = 96
CH = 106
E_PAD = B * CH * NW
NROWS = NW * CH
XROWS = 48
NBUF = 3
AHEAD = 2
FAST_CORE = 1
DB = 128
DCH = 80
DE_PAD = DB * DCH * NW
BR = 1024
GR = NP // BR


def _sc_mesh():
    return plsc.VectorSubcoreMesh(
        core_axis_name="c", subcore_axis_name="s", num_cores=NC, num_subcores=NS
    )


@functools.partial(
    pl.kernel,
    out_type=jax.ShapeDtypeStruct((NC, NP, DEGW), jnp.float32),
    mesh=_sc_mesh(),
    compiler_params=pltpu.CompilerParams(use_tc_tiling_on_sc=False),
    scratch_types=[
        pltpu.VMEM((DCH, DB), jnp.int32),
        pltpu.VMEM((DB, DEGW), jnp.float32),
        pltpu.VMEM_SHARED((NP, DEGW), jnp.float32),
    ],
)
def _deg_kernel(dst_hbm, out_hbm, dst_v, ones_v, acc):
    c = lax.axis_index("c")
    s = lax.axis_index("s")
    wid = s * NC + c
    base = s * RPT

    def fill(val):
        def body(i, carry):
            ones_v[i] = jnp.full((DEGW,), val, jnp.float32)
            return carry
        lax.fori_loop(0, DB, body, 0)

    fill(0.0)
    for t in range(RPT // DB):
        pltpu.sync_copy(ones_v, acc.at[pl.ds(base + t * DB, DB)])
    fill(1.0)
    plsc.subcore_barrier()

    pltpu.sync_copy(dst_hbm.at[pl.ds(wid * DCH, DCH)], dst_v)

    def chunk(j, carry):
        pltpu.sync_copy(ones_v, acc.at[dst_v.at[j]], add=True)
        return carry

    lax.fori_loop(0, DCH, chunk, 0)
    plsc.subcore_barrier()
    for t in range(RPT // DB):
        pltpu.sync_copy(
            acc.at[pl.ds(base + t * DB, DB)], out_hbm.at[c, pl.ds(base + t * DB, DB)]
        )


def _make_agg(F, chf, chs, qc):
    @functools.partial(
        pl.kernel,
        out_type=jax.ShapeDtypeStruct((NC, NP, F), jnp.float32),
        mesh=_sc_mesh(),
        compiler_params=pltpu.CompilerParams(use_tc_tiling_on_sc=False),
        scratch_types=[
            pltpu.VMEM((qc, B), jnp.int32),
            pltpu.VMEM((qc, B), jnp.int32),
            pltpu.VMEM((NBUF, B, F), jnp.float32),
            pltpu.VMEM_SHARED((NP, F), jnp.float32),
            pltpu.SemaphoreType.DMA,
        ],
    )
    def agg(h_hbm, src_hbm, dst_hbm, out_hbm, src_v, dst_v, rows_v, acc, sem):
        c = lax.axis_index("c")
        s = lax.axis_index("s")
        base = s * RPT
        is_fast = c == FAST_CORE
        n_ch = jnp.where(is_fast, chf, chs)
        row0 = jnp.where(is_fast, s * chf, 16 * chf + s * chs)

        def zero_rows(i, carry):
            for k in range(F // 16):
                rows_v[0, i, pl.ds(k * 16, 16)] = jnp.zeros((16,), jnp.float32)
            return carry

        lax.fori_loop(0, B, zero_rows, 0)
        nfull, tail = RPT // B, RPT % B
        for t in range(nfull):
            pltpu.sync_copy(rows_v.at[0], acc.at[pl.ds(base + t * B, B)])
        if tail:
            pltpu.sync_copy(
                rows_v.at[0].at[pl.ds(0, tail)], acc.at[pl.ds(base + nfull * B, tail)]
            )
        plsc.subcore_barrier()

        pltpu.sync_copy(src_hbm.at[pl.ds(row0, qc)], src_v)
        pltpu.sync_copy(dst_hbm.at[pl.ds(row0, qc)], dst_v)

        def start_gather(j):
            pltpu.async_copy(h_hbm.at[src_v.at[j % qc]], rows_v.at[j % NBUF], sem)

        for j in range(AHEAD):
            start_gather(j)

        def chunk(j, carry):
            slot = j % NBUF
            pltpu.make_async_copy(
                h_hbm.at[src_v.at[j % qc]], rows_v.at[slot], sem
            ).wait()

            nxt = j + AHEAD

            @pl.when((nxt < n_ch) & (nxt % qc == 0))
            def _():
                pltpu.sync_copy(src_hbm.at[pl.ds(row0 + nxt, qc)], src_v)

            @pl.when(nxt < n_ch)
            def _():
                start_gather(nxt)

            @pl.when((j % qc == 0) & (j > 0))
            def _():
                pltpu.sync_copy(dst_hbm.at[pl.ds(row0 + j, qc)], dst_v)

            pltpu.sync_copy(rows_v.at[slot], acc.at[dst_v.at[j % qc]], add=True)
            return carry

        lax.fori_loop(0, n_ch, chunk, 0)
        plsc.subcore_barrier()
        for t in range(nfull):
            pltpu.sync_copy(
                acc.at[pl.ds(base + t * B, B)], out_hbm.at[c, pl.ds(base + t * B, B)]
            )
        if tail:
            pltpu.sync_copy(
                acc.at[pl.ds(base + nfull * B, tail)],
                out_hbm.at[c, pl.ds(base + nfull * B, tail)],
            )

    return agg


_agg_l1 = _make_agg(F1, 168, 44, 42)
_agg_l2 = _make_agg(F2, 144, 68, 36)


def _dinv(deg_ref):
    d = deg_ref[0, :, :1] + deg_ref[1, :, :1] + 1.0
    return lax.rsqrt(d)


def _tc_pre1(degp, x_pad, W1):
    def body(deg_ref, x_ref, w_ref, o_ref):
        h = lax.dot_general(
            x_ref[...], w_ref[...], (((1,), (1,)), ((), ())),
            preferred_element_type=jnp.float32,
        )
        o_ref[...] = h * _dinv(deg_ref)

    return pl.pallas_call(
        body,
        grid=(GR,),
        in_specs=[
            pl.BlockSpec((NC, BR, DEGW), lambda i: (0, i, 0)),
            pl.BlockSpec((BR, F1), lambda i: (i, 0)),
            pl.BlockSpec((F1, F1), lambda i: (0, 0)),
        ],
        out_specs=pl.BlockSpec((BR, F1), lambda i: (i, 0)),
        out_shape=jax.ShapeDtypeStruct((NP, F1), jnp.float32),
    )(degp, x_pad, W1)


def _tc_mid(degp, p, h1, b1, W2):
    def body(deg_ref, p_ref, h1_ref, b1_ref, w_ref, o_ref):
        dinv = _dinv(deg_ref)
        g = (p_ref[0] + p_ref[1] + h1_ref[...]) * dinv + b1_ref[...]
        g = jnp.maximum(g, 0.0)
        h = lax.dot_general(
            g, w_ref[...], (((1,), (1,)), ((), ())),
            preferred_element_type=jnp.float32,
        )
        o_ref[...] = h * dinv

    return pl.pallas_call(
        body,
        grid=(GR,),
        in_specs=[
            pl.BlockSpec((NC, BR, DEGW), lambda i: (0, i, 0)),
            pl.BlockSpec((NC, BR, F1), lambda i: (0, i, 0)),
            pl.BlockSpec((BR, F1), lambda i: (i, 0)),
            pl.BlockSpec((1, F1), lambda i: (0, 0)),
            pl.BlockSpec((F2, F1), lambda i: (0, 0)),
        ],
        out_specs=pl.BlockSpec((BR, F2), lambda i: (i, 0)),
        out_shape=jax.ShapeDtypeStruct((NP, F2), jnp.float32),
    )(degp, p, h1, b1, W2)


def _tc_post(degp, q, h2, b2):
    def body(deg_ref, q_ref, h2_ref, b2_ref, o_ref):
        dinv = _dinv(deg_ref)
        g = (q_ref[0] + q_ref[1] + h2_ref[...]) * dinv + b2_ref[...]
        o_ref[...] = jnp.maximum(g, 0.0)

    return pl.pallas_call(
        body,
        grid=(GR,),
        in_specs=[
            pl.BlockSpec((NC, BR, DEGW), lambda i: (0, i, 0)),
            pl.BlockSpec((NC, BR, F2), lambda i: (0, i, 0)),
            pl.BlockSpec((BR, F2), lambda i: (i, 0)),
            pl.BlockSpec((1, F2), lambda i: (0, 0)),
        ],
        out_specs=pl.BlockSpec((BR, F2), lambda i: (i, 0)),
        out_shape=jax.ShapeDtypeStruct((NP, F2), jnp.float32),
    )(degp, q, h2, b2)


def kernel(x, adj, W1, b1, W2, b2):
    src = adj[0].astype(jnp.int32)
    dst = adj[1].astype(jnp.int32)
    pad = jnp.full((E_PAD + XROWS * B - E,), N, jnp.int32)
    srcp = jnp.concatenate([src, pad]).reshape(NROWS + XROWS, B)
    dstp = jnp.concatenate([dst, pad]).reshape(NROWS + XROWS, B)
    dpad = jnp.full((DE_PAD - E,), N, jnp.int32)
    dstp_deg = jnp.concatenate([dst, dpad]).reshape(NW * DCH, DB)
    x_pad = jnp.pad(x, ((0, NP - N), (0, 0)))

    degp = _deg_kernel(dstp_deg)
    h1 = _tc_pre1(degp, x_pad, W1)
    p = _agg_l1(h1, srcp, dstp)
    h2 = _tc_mid(degp, p, h1, b1.reshape(1, F1), W2)
    q = _agg_l2(h2, srcp, dstp)
    out = _tc_post(degp, q, h2, b2.reshape(1, F2))
    return out[:N]


# speedup vs baseline: 16.6208x; 1.0219x over previous
import functools

import jax
import jax.numpy as jnp
from jax import lax
from jax.experimental import pallas as pl
from jax.experimental.pallas import tpu as pltpu
from jax.experimental.pallas import tpu_sc as plsc

N = 10000
E = 320000
F1 = 128
F2 = 64
NC = 2
NS = 16
NW = NC * NS
NP = 10240
RPT = NP // NS
DEGW = 16
B = 96
CH = 106
E_PAD = B * CH * NW
NROWS = NW * CH
XROWS = 48
NBUF = 3
AHEAD = 2
FAST_CORE = 1
DB = 128
DCH = 80
DE_PAD = DB * DCH * NW
BR = 1024
GR = NP // BR


def _sc_mesh():
    return plsc.VectorSubcoreMesh(
        core_axis_name="c", subcore_axis_name="s", num_cores=NC, num_subcores=NS
    )


@functools.partial(
    pl.kernel,
    out_type=jax.ShapeDtypeStruct((NC, NP, DEGW), jnp.float32),
    mesh=_sc_mesh(),
    compiler_params=pltpu.CompilerParams(use_tc_tiling_on_sc=False),
    scratch_types=[
        pltpu.VMEM((DCH, DB), jnp.int32),
        pltpu.VMEM((DB, DEGW), jnp.float32),
        pltpu.VMEM_SHARED((NP, DEGW), jnp.float32),
    ],
)
def _deg_kernel(dst_hbm, out_hbm, dst_v, ones_v, acc):
    c = lax.axis_index("c")
    s = lax.axis_index("s")
    wid = s * NC + c
    base = s * RPT

    def fill(val):
        def body(i, carry):
            ones_v[i] = jnp.full((DEGW,), val, jnp.float32)
            return carry
        lax.fori_loop(0, DB, body, 0)

    fill(0.0)
    for t in range(RPT // DB):
        pltpu.sync_copy(ones_v, acc.at[pl.ds(base + t * DB, DB)])
    fill(1.0)
    plsc.subcore_barrier()

    pltpu.sync_copy(dst_hbm.at[pl.ds(wid * DCH, DCH)], dst_v)

    def chunk(j, carry):
        pltpu.sync_copy(ones_v, acc.at[dst_v.at[j]], add=True)
        return carry

    lax.fori_loop(0, DCH, chunk, 0)
    plsc.subcore_barrier()
    for t in range(RPT // DB):
        pltpu.sync_copy(
            acc.at[pl.ds(base + t * DB, DB)], out_hbm.at[c, pl.ds(base + t * DB, DB)]
        )


def _make_agg(F, chf, chs, qc):
    @functools.partial(
        pl.kernel,
        out_type=jax.ShapeDtypeStruct((NC, NP, F), jnp.float32),
        mesh=_sc_mesh(),
        compiler_params=pltpu.CompilerParams(use_tc_tiling_on_sc=False),
        scratch_types=[
            pltpu.VMEM((qc, B), jnp.int32),
            pltpu.VMEM((qc, B), jnp.int32),
            pltpu.VMEM((NBUF, B, F), jnp.float32),
            pltpu.VMEM_SHARED((NP, F), jnp.float32),
            pltpu.SemaphoreType.DMA,
        ],
    )
    def agg(h_hbm, src_hbm, dst_hbm, out_hbm, src_v, dst_v, rows_v, acc, sem):
        c = lax.axis_index("c")
        s = lax.axis_index("s")
        base = s * RPT
        is_fast = c == FAST_CORE
        n_ch = jnp.where(is_fast, chf, chs)
        row0 = jnp.where(is_fast, s * chf, 16 * chf + s * chs)

        def zero_rows(i, carry):
            for k in range(F // 16):
                rows_v[0, i, pl.ds(k * 16, 16)] = jnp.zeros((16,), jnp.float32)
            return carry

        lax.fori_loop(0, B, zero_rows, 0)
        nfull, tail = RPT // B, RPT % B
        for t in range(nfull):
            pltpu.sync_copy(rows_v.at[0], acc.at[pl.ds(base + t * B, B)])
        if tail:
            pltpu.sync_copy(
                rows_v.at[0].at[pl.ds(0, tail)], acc.at[pl.ds(base + nfull * B, tail)]
            )
        plsc.subcore_barrier()

        pltpu.sync_copy(src_hbm.at[pl.ds(row0, qc)], src_v)
        pltpu.sync_copy(dst_hbm.at[pl.ds(row0, qc)], dst_v)

        def start_gather(j):
            pltpu.async_copy(h_hbm.at[src_v.at[j % qc]], rows_v.at[j % NBUF], sem)

        for j in range(AHEAD):
            start_gather(j)

        def chunk(j, carry):
            slot = j % NBUF
            pltpu.make_async_copy(
                h_hbm.at[src_v.at[j % qc]], rows_v.at[slot], sem
            ).wait()

            nxt = j + AHEAD

            @pl.when((nxt < n_ch) & (nxt % qc == 0))
            def _():
                pltpu.sync_copy(src_hbm.at[pl.ds(row0 + nxt, qc)], src_v)

            @pl.when(nxt < n_ch)
            def _():
                start_gather(nxt)

            @pl.when((j % qc == 0) & (j > 0))
            def _():
                pltpu.sync_copy(dst_hbm.at[pl.ds(row0 + j, qc)], dst_v)

            pltpu.sync_copy(rows_v.at[slot], acc.at[dst_v.at[j % qc]], add=True)
            return carry

        lax.fori_loop(0, n_ch, chunk, 0)
        plsc.subcore_barrier()
        for t in range(nfull):
            pltpu.sync_copy(
                acc.at[pl.ds(base + t * B, B)], out_hbm.at[c, pl.ds(base + t * B, B)]
            )
        if tail:
            pltpu.sync_copy(
                acc.at[pl.ds(base + nfull * B, tail)],
                out_hbm.at[c, pl.ds(base + nfull * B, tail)],
            )

    return agg


_agg_l1 = _make_agg(F1, 188, 24, 47)
_agg_l2 = _make_agg(F2, 160, 52, 40)


def _dinv(deg_ref):
    d = deg_ref[0, :, :1] + deg_ref[1, :, :1] + 1.0
    return lax.rsqrt(d)


def _tc_pre1(degp, x_pad, W1):
    def body(deg_ref, x_ref, w_ref, o_ref):
        h = lax.dot_general(
            x_ref[...], w_ref[...], (((1,), (1,)), ((), ())),
            preferred_element_type=jnp.float32,
        )
        o_ref[...] = h * _dinv(deg_ref)

    return pl.pallas_call(
        body,
        grid=(GR,),
        in_specs=[
            pl.BlockSpec((NC, BR, DEGW), lambda i: (0, i, 0)),
            pl.BlockSpec((BR, F1), lambda i: (i, 0)),
            pl.BlockSpec((F1, F1), lambda i: (0, 0)),
        ],
        out_specs=pl.BlockSpec((BR, F1), lambda i: (i, 0)),
        out_shape=jax.ShapeDtypeStruct((NP, F1), jnp.float32),
    )(degp, x_pad, W1)


def _tc_mid(degp, p, h1, b1, W2):
    def body(deg_ref, p_ref, h1_ref, b1_ref, w_ref, o_ref):
        dinv = _dinv(deg_ref)
        g = (p_ref[0] + p_ref[1] + h1_ref[...]) * dinv + b1_ref[...]
        g = jnp.maximum(g, 0.0)
        h = lax.dot_general(
            g, w_ref[...], (((1,), (1,)), ((), ())),
            preferred_element_type=jnp.float32,
        )
        o_ref[...] = h * dinv

    return pl.pallas_call(
        body,
        grid=(GR,),
        in_specs=[
            pl.BlockSpec((NC, BR, DEGW), lambda i: (0, i, 0)),
            pl.BlockSpec((NC, BR, F1), lambda i: (0, i, 0)),
            pl.BlockSpec((BR, F1), lambda i: (i, 0)),
            pl.BlockSpec((1, F1), lambda i: (0, 0)),
            pl.BlockSpec((F2, F1), lambda i: (0, 0)),
        ],
        out_specs=pl.BlockSpec((BR, F2), lambda i: (i, 0)),
        out_shape=jax.ShapeDtypeStruct((NP, F2), jnp.float32),
    )(degp, p, h1, b1, W2)


def _tc_post(degp, q, h2, b2):
    def body(deg_ref, q_ref, h2_ref, b2_ref, o_ref):
        dinv = _dinv(deg_ref)
        g = (q_ref[0] + q_ref[1] + h2_ref[...]) * dinv + b2_ref[...]
        o_ref[...] = jnp.maximum(g, 0.0)

    return pl.pallas_call(
        body,
        grid=(GR,),
        in_specs=[
            pl.BlockSpec((NC, BR, DEGW), lambda i: (0, i, 0)),
            pl.BlockSpec((NC, BR, F2), lambda i: (0, i, 0)),
            pl.BlockSpec((BR, F2), lambda i: (i, 0)),
            pl.BlockSpec((1, F2), lambda i: (0, 0)),
        ],
        out_specs=pl.BlockSpec((BR, F2), lambda i: (i, 0)),
        out_shape=jax.ShapeDtypeStruct((NP, F2), jnp.float32),
    )(degp, q, h2, b2)


def kernel(x, adj, W1, b1, W2, b2):
    src = adj[0].astype(jnp.int32)
    dst = adj[1].astype(jnp.int32)
    pad = jnp.full((E_PAD + XROWS * B - E,), N, jnp.int32)
    srcp = jnp.concatenate([src, pad]).reshape(NROWS + XROWS, B)
    dstp = jnp.concatenate([dst, pad]).reshape(NROWS + XROWS, B)
    dpad = jnp.full((DE_PAD - E,), N, jnp.int32)
    dstp_deg = jnp.concatenate([dst, dpad]).reshape(NW * DCH, DB)
    x_pad = jnp.pad(x, ((0, NP - N), (0, 0)))

    degp = _deg_kernel(dstp_deg)
    h1 = _tc_pre1(degp, x_pad, W1)
    p = _agg_l1(h1, srcp, dstp)
    h2 = _tc_mid(degp, p, h1, b1.reshape(1, F1), W2)
    q = _agg_l2(h2, srcp, dstp)
    out = _tc_post(degp, q, h2, b2.reshape(1, F2))
    return out[:N]
